# baseline (speedup 1.0000x reference)
import functools

import jax
import jax.numpy as jnp
from jax import lax
from jax.experimental import pallas as pl
from jax.experimental.pallas import tpu as pltpu
from jax.experimental.pallas import tpu_sc as plsc

B = 16384
F = 16
NC = 2
NS = 16
NW = NC * NS
BPW = B // NW
WV = 16
NWAVES = BPW // WV


def _mf_body(ue_h, ie_h, ub_h, ib_h, uids_h, iids_h, out_h,
             uidx_v, iidx_v, ub0_v, ub1_v, iblk_v, ub_v, ib_v, o_v,
             su0, su1, si, s2, s3):
    wid = lax.axis_index("s") * NC + lax.axis_index("c")
    base = wid * BPW

    pltpu.sync_copy(uids_h.at[pl.ds(base, BPW)], uidx_v)
    pltpu.sync_copy(iids_h.at[pl.ds(base, BPW)], iidx_v)
    cub = pltpu.async_copy(ub_h.at[uidx_v], ub_v, s2)
    cib = pltpu.async_copy(ib_h.at[iidx_v], ib_v, s3)

    lanes = lax.iota(jnp.int32, 16)

    def fire_user(w, ubuf, sem):
        cols = lax.shift_right_logical(uidx_v[pl.ds(w * WV, WV)], 7) * 128
        for j in range(WV):
            c = pl.multiple_of(cols[j], 128)
            pltpu.async_copy(ue_h.at[:, pl.ds(c, 128)], ubuf.at[j], sem)

    def fire_item(w):
        cols = lax.shift_right_logical(iidx_v[pl.ds(w * WV, WV)], 7) * 128
        for j in range(WV):
            c = pl.multiple_of(cols[j], 128)
            pltpu.async_copy(ie_h.at[:, pl.ds(c, 128)], iblk_v.at[j], si)

    def drain(ubuf, sem):
        pltpu.make_async_copy(ue_h.at[:, pl.ds(0, WV * 128)], ubuf, sem).wait()

    def drain_item():
        pltpu.make_async_copy(ie_h.at[:, pl.ds(0, WV * 128)], iblk_v, si).wait()

    def extract(w, ubuf):
        sl = pl.ds(w * WV, WV)
        ucol = uidx_v[sl] & 127
        icol = iidx_v[sl] & 127
        acc = jnp.zeros((16,), jnp.float32)
        for f in range(F):
            fvec = jnp.full((16,), f, jnp.int32)
            uval = plsc.load_gather(ubuf, [lanes, fvec, ucol])
            ival = plsc.load_gather(iblk_v, [lanes, fvec, icol])
            acc = acc + uval * ival
        o_v[sl] = acc

    fire_user(0, ub0_v, su0)

    def pair(p, carry):
        w0 = p * 2
        w1 = w0 + 1
        fire_user(w1, ub1_v, su1)
        fire_item(w0)
        drain(ub0_v, su0)
        drain_item()
        extract(w0, ub0_v)

        @pl.when(p + 1 < NWAVES // 2)
        def _():
            fire_user(w0 + 2, ub0_v, su0)
        fire_item(w1)
        drain(ub1_v, su1)
        drain_item()
        extract(w1, ub1_v)
        return carry

    lax.fori_loop(0, NWAVES // 2, pair, 0)
    cub.wait()
    cib.wait()

    def addb(g, carry):
        sl = pl.ds(g * 16, 16)
        o_v[sl] = o_v[sl] + ub_v[sl] + ib_v[sl]
        return carry

    lax.fori_loop(0, BPW // 16, addb, 0)
    pltpu.sync_copy(o_v, out_h.at[pl.ds(base, BPW)])


@jax.jit
def _mf(uids, iids, user_embeddings, item_embeddings, user_bias, item_bias):
    mesh = plsc.VectorSubcoreMesh(core_axis_name="c", subcore_axis_name="s",
                                  num_cores=NC, num_subcores=NS)
    return pl.kernel(
        _mf_body,
        out_type=jax.ShapeDtypeStruct((B,), jnp.float32),
        mesh=mesh,
        compiler_params=pltpu.CompilerParams(
            needs_layout_passes=False, use_tc_tiling_on_sc=True),
        scratch_types=[
            pltpu.VMEM((BPW,), jnp.int32),
            pltpu.VMEM((BPW,), jnp.int32),
            pltpu.VMEM((WV, F, 128), jnp.float32),
            pltpu.VMEM((WV, F, 128), jnp.float32),
            pltpu.VMEM((WV, F, 128), jnp.float32),
            pltpu.VMEM((BPW,), jnp.float32),
            pltpu.VMEM((BPW,), jnp.float32),
            pltpu.VMEM((BPW,), jnp.float32),
            pltpu.SemaphoreType.DMA,
            pltpu.SemaphoreType.DMA,
            pltpu.SemaphoreType.DMA,
            pltpu.SemaphoreType.DMA,
            pltpu.SemaphoreType.DMA,
        ],
    )(user_embeddings.T, item_embeddings.T,
      user_bias.reshape(-1), item_bias.reshape(-1), uids, iids)


def kernel(uids, iids, user_embeddings, item_embeddings, user_bias, item_bias):
    return _mf(uids, iids, user_embeddings, item_embeddings,
               user_bias, item_bias)
